# baseline (speedup 1.0000x reference)
import functools

import jax
import jax.numpy as jnp
from jax.experimental import pallas as pl
from jax.experimental.pallas import tpu as pltpu
from jax.experimental.pallas import tpu_sc as plsc

KK = 20
NN = 2048
ROW_BLK = 256
_GW = 128


def _sc_gather(table, idx):
    t, c = table.shape
    m = idx.shape[0]
    half = m // 2
    steps = half // _GW
    idx3 = idx.reshape(2, steps, _GW)
    mesh = plsc.VectorSubcoreMesh(core_axis_name="core", subcore_axis_name="subcore")

    @functools.partial(
        pl.kernel,
        out_type=jax.ShapeDtypeStruct((m, c), jnp.float32),
        mesh=mesh,
        compiler_params=pltpu.CompilerParams(use_tc_tiling_on_sc=False),
    )
    def k(x_hbm, i_hbm, o_hbm):
        cid = jax.lax.axis_index("core")

        def body(i_vmem, o_vmem):
            pltpu.sync_copy(x_hbm.at[i_vmem.at[0]], o_vmem)

        pltpu.emit_pipeline(
            body,
            grid=(steps,),
            in_specs=[pl.BlockSpec((1, _GW), index_map=lambda i: (i, 0))],
            out_specs=[pl.BlockSpec((_GW, c), index_map=lambda i: (i, 0))],
            core_axis_name="subcore",
            dimension_semantics=(pltpu.PARALLEL,),
        )(i_hbm.at[cid], o_hbm.at[pl.ds(cid * half, half)])

    return k(table, idx3)


def _knn_body(xt_ref, x_ref, out_ref):
    xt = xt_ref[0]
    x = x_ref[0]
    g = jax.lax.dot_general(
        xt.astype(jnp.bfloat16), x.astype(jnp.bfloat16),
        (((1,), (0,)), ((), ())),
        preferred_element_type=jnp.float32,
    )
    xx_r = jnp.sum(xt * xt, axis=1, keepdims=True)
    xx_c = jnp.sum(x * x, axis=0, keepdims=True)
    pd = 2.0 * g - xx_r - xx_c
    iota = jax.lax.broadcasted_iota(jnp.int32, (ROW_BLK, NN), 1)
    ninf = jnp.float32(-jnp.inf)
    for j in range(KK):
        m = jnp.max(pd, axis=1, keepdims=True)
        am = jnp.min(jnp.where(pd == m, iota, jnp.int32(NN)), axis=1, keepdims=True)
        pd = jnp.where(iota == am, ninf, pd)
        out_ref[0, :, pl.ds(j, 1)] = am


def _knn_idx(x):
    b, c, n = x.shape
    cp = max(8, c)
    if cp != c:
        xp = jnp.zeros((b, cp, n), x.dtype).at[:, :c, :].set(x)
    else:
        xp = x
    xt = jnp.transpose(xp, (0, 2, 1))
    out = pl.pallas_call(
        _knn_body,
        grid=(b, n // ROW_BLK),
        in_specs=[
            pl.BlockSpec((1, ROW_BLK, cp), lambda i, j: (i, j, 0)),
            pl.BlockSpec((1, cp, n), lambda i, j: (i, 0, 0)),
        ],
        out_specs=pl.BlockSpec((1, ROW_BLK, KK), lambda i, j: (i, j, 0)),
        out_shape=jax.ShapeDtypeStruct((b, n, KK), jnp.int32),
    )(xt, xp)
    return out


NB = 256


def _dotbf(a, b):
    return jax.lax.dot_general(
        a.astype(jnp.bfloat16), b.astype(jnp.bfloat16),
        (((1,), (0,)), ((), ())), preferred_element_type=jnp.float32)


def _make_block_body(nlayers, b_sz, nblk, cin2, couts):
    npts = float(b_sz * nblk * NB * KK)

    def body(feat_ref, xc_ref, *rest):
        w_refs = rest[:nlayers]
        bn_ref = rest[nlayers]
        out_ref = rest[nlayers + 1]
        st_ref = rest[nlayers + 2]
        p = pl.program_id(0)
        bi = pl.program_id(1)
        nb = pl.program_id(2)
        first = (p == 0) & (bi == 0) & (nb == 0)
        last = (bi == b_sz - 1) & (nb == nblk - 1)

        @pl.when(first)
        def _():
            st_ref[...] = jnp.zeros_like(st_ref)

        f = feat_ref[0, :, 0]
        xc = xc_ref[0, 0]
        g = jnp.concatenate(
            [f - xc[None], jnp.broadcast_to(xc[None], f.shape)], axis=-1)
        h = g.reshape(KK * NB, cin2)

        for li in range(nlayers):
            h = _dotbf(h, w_refs[li][...])

            @pl.when(p == li)
            def _(h=h, li=li):
                st_ref[2 * li, :] += jnp.sum(h, axis=0)
                st_ref[2 * li + 1, :] += jnp.sum(h * h, axis=0)

                @pl.when(last)
                def _():
                    mean = st_ref[2 * li, :] / npts
                    var = st_ref[2 * li + 1, :] / npts - mean * mean
                    sc = bn_ref[2 * li, :] / jnp.sqrt(var + 1e-5)
                    st_ref[2 * nlayers + 2 * li, :] = sc
                    st_ref[2 * nlayers + 2 * li + 1, :] = (
                        bn_ref[2 * li + 1, :] - mean * sc)

            sc = st_ref[2 * nlayers + 2 * li, :]
            bi_ = st_ref[2 * nlayers + 2 * li + 1, :]
            h = h * sc[None] + bi_[None]
            h = jnp.where(h >= 0, h, 0.2 * h)

        @pl.when(p == nlayers)
        def _():
            hk = h.reshape(KK, NB, h.shape[-1])
            m = jnp.max(hk, axis=0)
            e = jnp.exp(hk - m[None])
            s = jnp.sum(e, axis=0)
            t = jnp.sum(e * hk, axis=0)
            out_ref[0, 0, 0] = t / s

    return body


def _fused_block(feat, xself_t, ws, bns):
    b, n, cin = xself_t.shape
    nlayers = len(ws)
    nblk = n // NB
    cmax = max(w.shape[1] for w in ws)
    cin2 = ws[0].shape[0]
    feat5 = feat.reshape(b, KK, nblk, NB, cin)
    xc4 = xself_t.reshape(b, nblk, NB, cin)
    wpads = []
    for i, w in enumerate(ws):
        rows = w.shape[0] if i == 0 else cmax
        if w.shape != (rows, cmax):
            wp = jnp.zeros((rows, cmax), w.dtype)
            wp = wp.at[:w.shape[0], :w.shape[1]].set(w)
        else:
            wp = w
        wpads.append(wp)
    bn_arr = jnp.zeros((2 * nlayers, cmax), jnp.float32)
    for i, (g_, b_) in enumerate(bns):
        bn_arr = bn_arr.at[2 * i, :g_.shape[0]].set(g_)
        bn_arr = bn_arr.at[2 * i + 1, :b_.shape[0]].set(b_)

    grid = (nlayers + 1, b, nblk)
    in_specs = [
        pl.BlockSpec((1, KK, 1, NB, cin), lambda p, bi, nb: (bi, 0, nb, 0, 0)),
        pl.BlockSpec((1, 1, NB, cin), lambda p, bi, nb: (bi, nb, 0, 0)),
    ]
    for w in wpads:
        in_specs.append(pl.BlockSpec(w.shape, lambda p, bi, nb: (0, 0)))
    in_specs.append(pl.BlockSpec(bn_arr.shape, lambda p, bi, nb: (0, 0)))
    out = pl.pallas_call(
        _make_block_body(nlayers, b, nblk, cin2, [w.shape[1] for w in ws]),
        grid=grid,
        in_specs=in_specs,
        out_specs=pl.BlockSpec((1, 1, 1, NB, cmax),
                               lambda p, bi, nb: (p, bi, nb, 0, 0)),
        out_shape=jax.ShapeDtypeStruct((nlayers + 1, b, nblk, NB, cmax),
                                       jnp.float32),
        scratch_shapes=[pltpu.VMEM((4 * nlayers, cmax), jnp.float32)],
    )(feat5, xc4, *wpads, bn_arr)
    cout = ws[-1].shape[1]
    return out[nlayers].reshape(b, n, cmax)[:, :, :cout]


def _edge_block(xin, conv_ws, conv_bns, p, pre):
    b, c, n = xin.shape
    cp = max(8, c)
    if cp != c:
        xp = jnp.zeros((b, cp, n), xin.dtype).at[:, :c, :].set(xin)
    else:
        xp = xin
    idx = _knn_idx(xin)
    xt = jnp.transpose(xp, (0, 2, 1))
    idx_kmaj = jnp.transpose(idx, (0, 2, 1))
    idx_flat = (idx_kmaj + (jnp.arange(b) * n)[:, None, None]).reshape(-1)
    feat = _sc_gather(xt.reshape(b * n, cp), idx_flat)
    w1 = conv_ws[0]
    w1t = jnp.zeros((2 * cp, w1.shape[0]), jnp.float32)
    w1t = w1t.at[:c].set(w1[:, :c].T).at[cp:cp + c].set(w1[:, c:].T)
    ws = [w1t] + [w.T for w in conv_ws[1:]]
    ws.append(p[pre + '_w1'].T)
    ws.append(p[pre + '_w2'].T)
    bns = list(conv_bns) + [
        (p[pre + '_g1'], p[pre + '_b1']),
        (p[pre + '_g2'], p[pre + '_b2']),
    ]
    a = _fused_block(feat, xt, ws, bns)
    return jnp.transpose(a, (0, 2, 1))


def _leaky(x):
    return jax.nn.leaky_relu(x, negative_slope=0.2)


def _bn(x, g, b, axes):
    m = jnp.mean(x, axis=axes, keepdims=True)
    v = jnp.var(x, axis=axes, keepdims=True)
    sh = [1] * x.ndim
    sh[1] = x.shape[1]
    return (x - m) / jnp.sqrt(v + 1e-5) * g.reshape(sh) + b.reshape(sh)


def _c2d(w, x):
    return jnp.einsum('oc,bcnk->bonk', w, x)


def _c1d(w, x):
    return jnp.einsum('oc,bcn->bon', w, x)


def _gather_rows(xt, idx_flat):
    t, c = xt.shape
    cp = max(8, c)
    if cp != c:
        xt = jnp.zeros((t, cp), xt.dtype).at[:, :c].set(xt)
    return _sc_gather(xt, idx_flat)[:, :c]


def _graph_feature(x, k):
    b, c, n = x.shape
    idx = _knn_idx(x)
    xt = jnp.transpose(x, (0, 2, 1)).reshape(b * n, c)
    idx_flat = (idx + (jnp.arange(b) * n)[:, None, None]).reshape(-1)
    feat = _gather_rows(xt, idx_flat).reshape(b, n, k, c)
    xc = jnp.broadcast_to(xt.reshape(b, n, 1, c), (b, n, k, c))
    out = jnp.concatenate([feat - xc, xc], axis=3)
    return jnp.transpose(out, (0, 3, 1, 2)), idx_flat


def _car2sph(x):
    x = jnp.transpose(x, (0, 2, 3, 1))
    xsq = x[..., 0] ** 2 + x[..., 1] ** 2
    r = jnp.sqrt(xsq + x[..., 2] ** 2)
    th = jnp.arctan2(x[..., 2], jnp.sqrt(xsq))
    ph = jnp.arctan2(x[..., 1], x[..., 0])
    sph = jnp.stack([r, th, ph], axis=-1)
    avg = jnp.mean(sph, axis=-1, keepdims=True)
    cat = jnp.concatenate([sph, sph - avg], axis=-1)
    return jnp.transpose(cat, (0, 3, 1, 2))


def _transform_net(x0, p):
    b = x0.shape[0]
    h = _leaky(_bn(_c2d(p['tw1'], x0), p['tg1'], p['tb1'], (0, 2, 3)))
    h = _leaky(_bn(_c2d(p['tw2'], h), p['tg2'], p['tb2'], (0, 2, 3)))
    h = jnp.max(h, axis=-1)
    h = _leaky(_bn(_c1d(p['tw3'], h), p['tg3'], p['tb3'], (0, 2)))
    h = jnp.max(h, axis=-1)
    h = _leaky(_bn(h @ p['tl1'].T, p['tg4'], p['tb4'], (0,)))
    h = _leaky(_bn(h @ p['tl2'].T, p['tg5'], p['tb5'], (0,)))
    t = h @ p['ttw'].T + p['ttb']
    return t.reshape(b, 3, 3)


def _attpool(x, p, pre):
    h = _leaky(_bn(_c2d(p[pre + '_w1'], x), p[pre + '_g1'], p[pre + '_b1'], (0, 2, 3)))
    h = _leaky(_bn(_c2d(p[pre + '_w2'], h), p[pre + '_g2'], p[pre + '_b2'], (0, 2, 3)))
    att = jax.nn.softmax(h, axis=-1)
    return jnp.sum(att * h, axis=-1)


def _sfp(x_loc, x, p, k):
    b, c, n = x.shape
    gf, idx = _graph_feature(x_loc, k)
    loc = _car2sph(gf[:, :3, :, :])
    loc = jnp.transpose(loc, (0, 2, 1, 3)).reshape(b * n, 6, k)
    h = _leaky(_bn(_c1d(p['s_w1'], loc), p['s_g1'], p['s_b1'], (0, 2)))
    h = _leaky(_bn(_c1d(p['s_w2'], h), p['s_g2'], p['s_b2'], (0, 2)))
    h = _leaky(_bn(_c1d(p['s_w3'], h), p['s_g3'], p['s_b3'], (0, 2)))
    att = jax.nn.softmax(h.reshape(b, n, 1, k), axis=-1)
    att = jnp.broadcast_to(att, (b, n, c, k))
    att = jnp.transpose(att, (0, 2, 1, 3))
    xk = _gather_rows(jnp.transpose(x, (0, 2, 1)).reshape(b * n, c), idx).reshape(b, n, k, c)
    xk = jnp.transpose(xk, (0, 3, 1, 2))
    return x + jnp.sum(xk * att, axis=-1)


def _forward(x, l, p):
    b, _, n = x.shape
    x0, _ = _graph_feature(x, KK)
    t = _transform_net(x0, p)
    x = jnp.transpose(jnp.matmul(jnp.transpose(x, (0, 2, 1)), t), (0, 2, 1))
    x_loc = x
    g, _ = _graph_feature(x, KK)
    h = _leaky(_bn(_c2d(p['w1'], g), p['g1'], p['b1'], (0, 2, 3)))
    h = _leaky(_bn(_c2d(p['w2'], h), p['g2'], p['b2'], (0, 2, 3)))
    x1 = _sfp(x_loc, _attpool(h, p, 'p1'), p, KK)
    g, _ = _graph_feature(x1, KK)
    h = _leaky(_bn(_c2d(p['w3'], g), p['g3'], p['b3'], (0, 2, 3)))
    h = _leaky(_bn(_c2d(p['w4'], h), p['g4'], p['b4'], (0, 2, 3)))
    x2 = _sfp(x_loc, _attpool(h, p, 'p2'), p, KK)
    a3 = _edge_block(x2, [p['w5']], [(p['g5'], p['b5'])], p, 'p3')
    x3 = _sfp(x_loc, a3, p, KK)
    xc = jnp.concatenate([x1, x2, x3], axis=1)
    e = jnp.max(_leaky(_bn(_c1d(p['w6'], xc), p['g6'], p['b6'], (0, 2))), axis=-1)
    lv = _leaky(_bn(_c1d(p['w7'], l), p['g7'], p['b7'], (0, 2)))
    glob = jnp.concatenate([e[:, :, None], lv], axis=1)
    glob = jnp.broadcast_to(glob, (b, glob.shape[1], n))
    f = jnp.concatenate([glob, x1, x2, x3], axis=1)
    f = _leaky(_bn(_c1d(p['w8'], f), p['g8'], p['b8'], (0, 2)))
    f = _leaky(_bn(_c1d(p['w9'], f), p['g9'], p['b9'], (0, 2)))
    f = _leaky(_bn(_c1d(p['w10'], f), p['g10'], p['b10'], (0, 2)))
    return _c1d(p['w11'], f)


def kernel(x, l, params):
    return _forward(x, l, params)


# speedup vs baseline: 1.0982x; 1.0982x over previous
import functools

import jax
import jax.numpy as jnp
from jax.experimental import pallas as pl
from jax.experimental.pallas import tpu as pltpu
from jax.experimental.pallas import tpu_sc as plsc

KK = 20
NN = 2048
ROW_BLK = 512
_GW = 128


def _sc_gather(table, idx):
    t, c = table.shape
    m = idx.shape[0]
    half = m // 2
    steps = half // _GW
    idx3 = idx.reshape(2, steps, _GW)
    mesh = plsc.VectorSubcoreMesh(core_axis_name="core", subcore_axis_name="subcore")

    @functools.partial(
        pl.kernel,
        out_type=jax.ShapeDtypeStruct((m, c), jnp.float32),
        mesh=mesh,
        compiler_params=pltpu.CompilerParams(use_tc_tiling_on_sc=False),
    )
    def k(x_hbm, i_hbm, o_hbm):
        cid = jax.lax.axis_index("core")

        def body(i_vmem, o_vmem):
            pltpu.sync_copy(x_hbm.at[i_vmem.at[0]], o_vmem)

        pltpu.emit_pipeline(
            body,
            grid=(steps,),
            in_specs=[pl.BlockSpec((1, _GW), index_map=lambda i: (i, 0))],
            out_specs=[pl.BlockSpec((_GW, c), index_map=lambda i: (i, 0))],
            core_axis_name="subcore",
            dimension_semantics=(pltpu.PARALLEL,),
        )(i_hbm.at[cid], o_hbm.at[pl.ds(cid * half, half)])

    return k(table, idx3)


def _knn_body(xt_ref, x_ref, out_ref):
    xt = xt_ref[0]
    x = x_ref[0]
    g = jax.lax.dot_general(
        xt.astype(jnp.bfloat16), x.astype(jnp.bfloat16),
        (((1,), (0,)), ((), ())),
        preferred_element_type=jnp.float32,
    )
    xx_r = jnp.sum(xt * xt, axis=1, keepdims=True)
    xx_c = jnp.sum(x * x, axis=0, keepdims=True)
    pd = 2.0 * g - xx_r - xx_c
    iota = jax.lax.broadcasted_iota(jnp.int32, (ROW_BLK, NN), 1)
    ninf = jnp.float32(-jnp.inf)
    for j in range(KK):
        m = jnp.max(pd, axis=1, keepdims=True)
        am = jnp.min(jnp.where(pd == m, iota, jnp.int32(NN)), axis=1, keepdims=True)
        pd = jnp.where(iota == am, ninf, pd)
        out_ref[0, :, pl.ds(j, 1)] = am


def _knn_idx(x):
    b, c, n = x.shape
    cp = max(8, c)
    if cp != c:
        xp = jnp.zeros((b, cp, n), x.dtype).at[:, :c, :].set(x)
    else:
        xp = x
    xt = jnp.transpose(xp, (0, 2, 1))
    out = pl.pallas_call(
        _knn_body,
        grid=(b, n // ROW_BLK),
        in_specs=[
            pl.BlockSpec((1, ROW_BLK, cp), lambda i, j: (i, j, 0)),
            pl.BlockSpec((1, cp, n), lambda i, j: (i, 0, 0)),
        ],
        out_specs=pl.BlockSpec((1, ROW_BLK, KK), lambda i, j: (i, j, 0)),
        out_shape=jax.ShapeDtypeStruct((b, n, KK), jnp.int32),
    )(xt, xp)
    return out


def _leaky(x):
    return jax.nn.leaky_relu(x, negative_slope=0.2)


def _bn(x, g, b, axes):
    m = jnp.mean(x, axis=axes, keepdims=True)
    v = jnp.var(x, axis=axes, keepdims=True)
    sh = [1] * x.ndim
    sh[1] = x.shape[1]
    return (x - m) / jnp.sqrt(v + 1e-5) * g.reshape(sh) + b.reshape(sh)


def _c2d(w, x):
    return jnp.einsum('oc,bcnk->bonk', w, x)


def _c1d(w, x):
    return jnp.einsum('oc,bcn->bon', w, x)


def _gather_rows(xt, idx_flat):
    t, c = xt.shape
    cp = max(8, c)
    if cp != c:
        xt = jnp.zeros((t, cp), xt.dtype).at[:, :c].set(xt)
    return _sc_gather(xt, idx_flat)[:, :c]


def _graph_feature(x, k):
    b, c, n = x.shape
    idx = _knn_idx(x)
    xt = jnp.transpose(x, (0, 2, 1)).reshape(b * n, c)
    idx_flat = (idx + (jnp.arange(b) * n)[:, None, None]).reshape(-1)
    feat = _gather_rows(xt, idx_flat).reshape(b, n, k, c)
    xc = jnp.broadcast_to(xt.reshape(b, n, 1, c), (b, n, k, c))
    out = jnp.concatenate([feat - xc, xc], axis=3)
    return jnp.transpose(out, (0, 3, 1, 2)), idx_flat


def _car2sph(x):
    x = jnp.transpose(x, (0, 2, 3, 1))
    xsq = x[..., 0] ** 2 + x[..., 1] ** 2
    r = jnp.sqrt(xsq + x[..., 2] ** 2)
    th = jnp.arctan2(x[..., 2], jnp.sqrt(xsq))
    ph = jnp.arctan2(x[..., 1], x[..., 0])
    sph = jnp.stack([r, th, ph], axis=-1)
    avg = jnp.mean(sph, axis=-1, keepdims=True)
    cat = jnp.concatenate([sph, sph - avg], axis=-1)
    return jnp.transpose(cat, (0, 3, 1, 2))


def _transform_net(x0, p):
    b = x0.shape[0]
    h = _leaky(_bn(_c2d(p['tw1'], x0), p['tg1'], p['tb1'], (0, 2, 3)))
    h = _leaky(_bn(_c2d(p['tw2'], h), p['tg2'], p['tb2'], (0, 2, 3)))
    h = jnp.max(h, axis=-1)
    h = _leaky(_bn(_c1d(p['tw3'], h), p['tg3'], p['tb3'], (0, 2)))
    h = jnp.max(h, axis=-1)
    h = _leaky(_bn(h @ p['tl1'].T, p['tg4'], p['tb4'], (0,)))
    h = _leaky(_bn(h @ p['tl2'].T, p['tg5'], p['tb5'], (0,)))
    t = h @ p['ttw'].T + p['ttb']
    return t.reshape(b, 3, 3)


def _attpool(x, p, pre):
    h = _leaky(_bn(_c2d(p[pre + '_w1'], x), p[pre + '_g1'], p[pre + '_b1'], (0, 2, 3)))
    h = _leaky(_bn(_c2d(p[pre + '_w2'], h), p[pre + '_g2'], p[pre + '_b2'], (0, 2, 3)))
    att = jax.nn.softmax(h, axis=-1)
    return jnp.sum(att * h, axis=-1)


def _sfp(x_loc, x, p, k):
    b, c, n = x.shape
    gf, idx = _graph_feature(x_loc, k)
    loc = _car2sph(gf[:, :3, :, :])
    loc = jnp.transpose(loc, (0, 2, 1, 3)).reshape(b * n, 6, k)
    h = _leaky(_bn(_c1d(p['s_w1'], loc), p['s_g1'], p['s_b1'], (0, 2)))
    h = _leaky(_bn(_c1d(p['s_w2'], h), p['s_g2'], p['s_b2'], (0, 2)))
    h = _leaky(_bn(_c1d(p['s_w3'], h), p['s_g3'], p['s_b3'], (0, 2)))
    att = jax.nn.softmax(h.reshape(b, n, 1, k), axis=-1)
    att = jnp.broadcast_to(att, (b, n, c, k))
    att = jnp.transpose(att, (0, 2, 1, 3))
    xk = _gather_rows(jnp.transpose(x, (0, 2, 1)).reshape(b * n, c), idx).reshape(b, n, k, c)
    xk = jnp.transpose(xk, (0, 3, 1, 2))
    return x + jnp.sum(xk * att, axis=-1)


def _forward(x, l, p):
    b, _, n = x.shape
    x0, _ = _graph_feature(x, KK)
    t = _transform_net(x0, p)
    x = jnp.transpose(jnp.matmul(jnp.transpose(x, (0, 2, 1)), t), (0, 2, 1))
    x_loc = x
    g, _ = _graph_feature(x, KK)
    h = _leaky(_bn(_c2d(p['w1'], g), p['g1'], p['b1'], (0, 2, 3)))
    h = _leaky(_bn(_c2d(p['w2'], h), p['g2'], p['b2'], (0, 2, 3)))
    x1 = _sfp(x_loc, _attpool(h, p, 'p1'), p, KK)
    g, _ = _graph_feature(x1, KK)
    h = _leaky(_bn(_c2d(p['w3'], g), p['g3'], p['b3'], (0, 2, 3)))
    h = _leaky(_bn(_c2d(p['w4'], h), p['g4'], p['b4'], (0, 2, 3)))
    x2 = _sfp(x_loc, _attpool(h, p, 'p2'), p, KK)
    g, _ = _graph_feature(x2, KK)
    h = _leaky(_bn(_c2d(p['w5'], g), p['g5'], p['b5'], (0, 2, 3)))
    x3 = _sfp(x_loc, _attpool(h, p, 'p3'), p, KK)
    xc = jnp.concatenate([x1, x2, x3], axis=1)
    e = jnp.max(_leaky(_bn(_c1d(p['w6'], xc), p['g6'], p['b6'], (0, 2))), axis=-1)
    lv = _leaky(_bn(_c1d(p['w7'], l), p['g7'], p['b7'], (0, 2)))
    glob = jnp.concatenate([e[:, :, None], lv], axis=1)
    glob = jnp.broadcast_to(glob, (b, glob.shape[1], n))
    f = jnp.concatenate([glob, x1, x2, x3], axis=1)
    f = _leaky(_bn(_c1d(p['w8'], f), p['g8'], p['b8'], (0, 2)))
    f = _leaky(_bn(_c1d(p['w9'], f), p['g9'], p['b9'], (0, 2)))
    f = _leaky(_bn(_c1d(p['w10'], f), p['g10'], p['b10'], (0, 2)))
    return _c1d(p['w11'], f)


def kernel(x, l, params):
    return _forward(x, l, params)
